# baseline (speedup 1.0000x reference)
import functools

import jax
import jax.numpy as jnp
from jax import lax
from jax.experimental import pallas as pl
from jax.experimental.pallas import tpu as pltpu
from jax.experimental.pallas import tpu_sc as plsc

_HIDDEN = 1152
_BATCH = 16384
_NUM_CORES = 2
_NUM_SUBCORES = 16
_NW = _NUM_CORES * _NUM_SUBCORES
_ROWS_PER_W = _BATCH // _NW
_REP = 2
_N_OUT = _ROWS_PER_W // _REP


@functools.partial(
    pl.kernel,
    out_type=jax.ShapeDtypeStruct((_BATCH, _HIDDEN), jnp.float32),
    mesh=plsc.VectorSubcoreMesh(core_axis_name="c", subcore_axis_name="s"),
    scratch_types=[
        pltpu.VMEM((_REP, _HIDDEN), jnp.float32),
        pltpu.SemaphoreType.DMA,
    ],
)
def _broadcast_row(table_hbm, out_hbm, buf, sem):
    wid = lax.axis_index("s") * _NUM_CORES + lax.axis_index("c")
    pltpu.sync_copy(table_hbm.at[0], buf.at[0])

    def _fill_row(r, carry):
        for c in range(_HIDDEN // 16):
            buf[r, pl.ds(c * 16, 16)] = buf[0, pl.ds(c * 16, 16)]
        return carry

    lax.fori_loop(1, _REP, _fill_row, 0)
    base = wid * _ROWS_PER_W
    copies = [
        pltpu.async_copy(buf, out_hbm.at[pl.ds(base + i * _REP, _REP)], sem)
        for i in range(_N_OUT)
    ]
    for c in copies:
        c.wait()


def kernel(labels, table):
    del labels
    return _broadcast_row(table)
